# baseline (speedup 1.0000x reference)
import jax
import jax.numpy as jnp
from jax.experimental import pallas as pl
from jax.experimental.pallas import tpu as pltpu

_N = 2562
_CHUNK = 1288
_DMA_CHUNK = 1288


def _pad128(d):
    return ((d + 127) // 128) * 128


def _chunks():
    out = []
    off = 0
    while off < _N:
        out.append((off, min(_CHUNK, _N - off)))
        off += _CHUNK
    return out


def _elu(v):
    return jnp.where(v > 0, v, jnp.exp(jnp.minimum(v, 0.0)) - 1.0)


def _encoder_kernel(*refs):
    pos_ref, adj_hbm = refs[0], refs[1]
    n_layers = (len(refs) - 7) // 2
    w_refs = refs[2:2 + n_layers]
    b_refs = refs[2 + n_layers:2 + 2 * n_layers]
    out_ref = refs[2 + 2 * n_layers]
    adj32 = refs[-4]
    bufs = (refs[-3], refs[-2])
    sems = refs[-1]

    dims = [w.shape for w in w_refs]
    reassoc = [False for _ in dims]
    chunks = _chunks()

    dma_chunks = []
    off = 0
    while off < _N:
        dma_chunks.append((off, min(_DMA_CHUNK, _N - off)))
        off += _DMA_CHUNK
    cps = []
    for r, (off, sz) in enumerate(dma_chunks):
        cp = pltpu.make_async_copy(
            adj_hbm.at[pl.ds(off, sz), :],
            adj32.at[pl.ds(off, sz), :],
            sems.at[r])
        cp.start()
        cps.append(cp)

    s0 = jnp.dot(pos_ref[...], w_refs[0][...],
                 preferred_element_type=jnp.float32)
    b0 = b_refs[0][...]

    for r, (off, sz) in enumerate(dma_chunks):
        cps[r].wait()
        a_r = adj32[pl.ds(off, sz), :]
        agg = jnp.dot(a_r, s0, preferred_element_type=jnp.float32)
        xr = _elu(agg + b0)
        if reassoc[1]:
            bufs[1][pl.ds(off, sz), 0:dims[0][1]] = xr
        else:
            s_next = jnp.dot(xr, w_refs[1][...],
                             preferred_element_type=jnp.float32)
            bufs[1][pl.ds(off, sz), 0:dims[1][1]] = s_next

    acc = None
    for i in range(1, n_layers):
        src, dst = bufs[i % 2], bufs[(i + 1) % 2]
        din, dout = dims[i]
        b = b_refs[i][...]
        in_w = din if reassoc[i] else dout
        carry = src[:, 0:in_w]
        w_i = w_refs[i][...]
        if i + 1 < n_layers:
            w_next = w_refs[i + 1][...]
        for off, sz in chunks:
            a_r = adj32[pl.ds(off, sz), :]
            if reassoc[i]:
                h = jnp.dot(a_r, carry, preferred_element_type=jnp.float32)
                agg = jnp.dot(h, w_i, preferred_element_type=jnp.float32)
            else:
                agg = jnp.dot(a_r, carry, preferred_element_type=jnp.float32)
            xr = _elu(agg + b)
            if i + 1 < n_layers:
                if reassoc[i + 1]:
                    dst[pl.ds(off, sz), 0:dout] = xr
                else:
                    s_next = jnp.dot(xr, w_next,
                                     preferred_element_type=jnp.float32)
                    dst[pl.ds(off, sz), 0:dims[i + 1][1]] = s_next
            else:
                m = jnp.max(xr, axis=0, keepdims=True)
                acc = m if acc is None else jnp.maximum(acc, m)
    out_ref[...] = acc


def kernel(positions, adj, Ws, bs):
    bs2 = [b.reshape(1, -1) for b in bs]
    max_w = max(max(d) for d in (w.shape for w in Ws))
    n_in = 2 + len(Ws) + len(bs)
    in_specs = [pl.BlockSpec(memory_space=pltpu.MemorySpace.HBM) if i == 1
                else pl.BlockSpec(memory_space=pltpu.MemorySpace.VMEM)
                for i in range(n_in)]
    out = pl.pallas_call(
        _encoder_kernel,
        out_shape=jax.ShapeDtypeStruct((1, Ws[-1].shape[1]), jnp.float32),
        in_specs=in_specs,
        out_specs=pl.BlockSpec(memory_space=pltpu.MemorySpace.VMEM),
        scratch_shapes=[
            pltpu.VMEM((_N, _N), jnp.float32),
            pltpu.VMEM((_N, _pad128(max_w)), jnp.float32),
            pltpu.VMEM((_N, _pad128(max_w)), jnp.float32),
            pltpu.SemaphoreType.DMA(((_N + _DMA_CHUNK - 1) // _DMA_CHUNK,)),
        ],
        compiler_params=pltpu.CompilerParams(
            vmem_limit_bytes=128 * 1024 * 1024,
        ),
    )(positions, adj, *Ws, *bs2)
    return out.reshape(-1)


# speedup vs baseline: 1.0132x; 1.0132x over previous
import jax
import jax.numpy as jnp
from jax.experimental import pallas as pl
from jax.experimental.pallas import tpu as pltpu

_N = 2562
_CHUNK = 1296
_DMA_CHUNK = 1296


def _pad128(d):
    return ((d + 127) // 128) * 128


def _chunks():
    out = []
    off = 0
    while off < _N:
        out.append((off, min(_CHUNK, _N - off)))
        off += _CHUNK
    return out


def _elu(v):
    return jnp.where(v > 0, v, jnp.exp(jnp.minimum(v, 0.0)) - 1.0)


def _encoder_kernel(*refs):
    pos_ref, adj_hbm = refs[0], refs[1]
    n_layers = (len(refs) - 7) // 2
    w_refs = refs[2:2 + n_layers]
    b_refs = refs[2 + n_layers:2 + 2 * n_layers]
    out_ref = refs[2 + 2 * n_layers]
    adj32 = refs[-4]
    bufs = (refs[-3], refs[-2])
    sems = refs[-1]

    dims = [w.shape for w in w_refs]
    reassoc = [_pad128(din) < _pad128(dout) for din, dout in dims]
    chunks = _chunks()

    dma_chunks = []
    off = 0
    while off < _N:
        dma_chunks.append((off, min(_DMA_CHUNK, _N - off)))
        off += _DMA_CHUNK
    cps = []
    for r, (off, sz) in enumerate(dma_chunks):
        cp = pltpu.make_async_copy(
            adj_hbm.at[pl.ds(off, sz), :],
            adj32.at[pl.ds(off, sz), :],
            sems.at[r])
        cp.start()
        cps.append(cp)

    s0 = jnp.dot(pos_ref[...], w_refs[0][...],
                 preferred_element_type=jnp.float32)
    b0 = b_refs[0][...]

    for r, (off, sz) in enumerate(dma_chunks):
        cps[r].wait()
        a_r = adj32[pl.ds(off, sz), :]
        agg = jnp.dot(a_r, s0, preferred_element_type=jnp.float32)
        xr = _elu(agg + b0)
        if reassoc[1]:
            bufs[1][pl.ds(off, sz), 0:dims[0][1]] = xr
        else:
            s_next = jnp.dot(xr, w_refs[1][...],
                             preferred_element_type=jnp.float32)
            bufs[1][pl.ds(off, sz), 0:dims[1][1]] = s_next

    acc = None
    for i in range(1, n_layers):
        src, dst = bufs[i % 2], bufs[(i + 1) % 2]
        din, dout = dims[i]
        b = b_refs[i][...]
        in_w = din if reassoc[i] else dout
        carry = src[:, 0:in_w]
        w_i = w_refs[i][...]
        if i + 1 < n_layers:
            w_next = w_refs[i + 1][...]
        for off, sz in chunks:
            a_r = adj32[pl.ds(off, sz), :]
            if reassoc[i]:
                h = jnp.dot(a_r, carry, preferred_element_type=jnp.float32)
                agg = jnp.dot(h, w_i, preferred_element_type=jnp.float32)
            else:
                agg = jnp.dot(a_r, carry, preferred_element_type=jnp.float32)
            xr = _elu(agg + b)
            if i + 1 < n_layers:
                if reassoc[i + 1]:
                    dst[pl.ds(off, sz), 0:dout] = xr
                else:
                    s_next = jnp.dot(xr, w_next,
                                     preferred_element_type=jnp.float32)
                    dst[pl.ds(off, sz), 0:dims[i + 1][1]] = s_next
            else:
                m = jnp.max(xr, axis=0, keepdims=True)
                acc = m if acc is None else jnp.maximum(acc, m)
    out_ref[...] = acc


def kernel(positions, adj, Ws, bs):
    bs2 = [b.reshape(1, -1) for b in bs]
    max_w = max(max(d) for d in (w.shape for w in Ws))
    n_in = 2 + len(Ws) + len(bs)
    in_specs = [pl.BlockSpec(memory_space=pltpu.MemorySpace.HBM) if i == 1
                else pl.BlockSpec(memory_space=pltpu.MemorySpace.VMEM)
                for i in range(n_in)]
    out = pl.pallas_call(
        _encoder_kernel,
        out_shape=jax.ShapeDtypeStruct((1, Ws[-1].shape[1]), jnp.float32),
        in_specs=in_specs,
        out_specs=pl.BlockSpec(memory_space=pltpu.MemorySpace.VMEM),
        scratch_shapes=[
            pltpu.VMEM((_N, _N), jnp.float32),
            pltpu.VMEM((_N, _pad128(max_w)), jnp.float32),
            pltpu.VMEM((_N, _pad128(max_w)), jnp.float32),
            pltpu.SemaphoreType.DMA(((_N + _DMA_CHUNK - 1) // _DMA_CHUNK,)),
        ],
        compiler_params=pltpu.CompilerParams(
            vmem_limit_bytes=128 * 1024 * 1024,
        ),
    )(positions, adj, *Ws, *bs2)
    return out.reshape(-1)


# speedup vs baseline: 1.0179x; 1.0046x over previous
import jax
import jax.numpy as jnp
from jax.experimental import pallas as pl
from jax.experimental.pallas import tpu as pltpu

_N = 2562
_CHUNK = 1304
_DMA_CHUNK = 1304


def _pad128(d):
    return ((d + 127) // 128) * 128


def _chunks():
    out = []
    off = 0
    while off < _N:
        out.append((off, min(_CHUNK, _N - off)))
        off += _CHUNK
    return out


def _elu(v):
    return jnp.where(v > 0, v, jnp.exp(jnp.minimum(v, 0.0)) - 1.0)


def _encoder_kernel(*refs):
    pos_ref, adj_hbm = refs[0], refs[1]
    n_layers = (len(refs) - 7) // 2
    w_refs = refs[2:2 + n_layers]
    b_refs = refs[2 + n_layers:2 + 2 * n_layers]
    out_ref = refs[2 + 2 * n_layers]
    adj32 = refs[-4]
    bufs = (refs[-3], refs[-2])
    sems = refs[-1]

    dims = [w.shape for w in w_refs]
    reassoc = [_pad128(din) < _pad128(dout) for din, dout in dims]
    chunks = _chunks()

    dma_chunks = []
    off = 0
    while off < _N:
        dma_chunks.append((off, min(_DMA_CHUNK, _N - off)))
        off += _DMA_CHUNK
    cps = []
    for r, (off, sz) in enumerate(dma_chunks):
        cp = pltpu.make_async_copy(
            adj_hbm.at[pl.ds(off, sz), :],
            adj32.at[pl.ds(off, sz), :],
            sems.at[r])
        cp.start()
        cps.append(cp)

    s0 = jnp.dot(pos_ref[...], w_refs[0][...],
                 preferred_element_type=jnp.float32)
    b0 = b_refs[0][...]

    for r, (off, sz) in enumerate(dma_chunks):
        cps[r].wait()
        a_r = adj32[pl.ds(off, sz), :]
        agg = jnp.dot(a_r, s0, preferred_element_type=jnp.float32)
        xr = _elu(agg + b0)
        if reassoc[1]:
            bufs[1][pl.ds(off, sz), 0:dims[0][1]] = xr
        else:
            s_next = jnp.dot(xr, w_refs[1][...],
                             preferred_element_type=jnp.float32)
            bufs[1][pl.ds(off, sz), 0:dims[1][1]] = s_next

    acc = None
    for i in range(1, n_layers):
        src, dst = bufs[i % 2], bufs[(i + 1) % 2]
        din, dout = dims[i]
        b = b_refs[i][...]
        in_w = din if reassoc[i] else dout
        carry = src[:, 0:in_w]
        w_i = w_refs[i][...]
        if i + 1 < n_layers:
            w_next = w_refs[i + 1][...]
        for off, sz in chunks:
            a_r = adj32[pl.ds(off, sz), :]
            if reassoc[i]:
                h = jnp.dot(a_r, carry, preferred_element_type=jnp.float32)
                agg = jnp.dot(h, w_i, preferred_element_type=jnp.float32)
            else:
                agg = jnp.dot(a_r, carry, preferred_element_type=jnp.float32)
            xr = _elu(agg + b)
            if i + 1 < n_layers:
                if reassoc[i + 1]:
                    dst[pl.ds(off, sz), 0:dout] = xr
                else:
                    s_next = jnp.dot(xr, w_next,
                                     preferred_element_type=jnp.float32)
                    dst[pl.ds(off, sz), 0:dims[i + 1][1]] = s_next
            else:
                m = jnp.max(xr, axis=0, keepdims=True)
                acc = m if acc is None else jnp.maximum(acc, m)
    out_ref[...] = acc


def kernel(positions, adj, Ws, bs):
    bs2 = [b.reshape(1, -1) for b in bs]
    max_w = max(max(d) for d in (w.shape for w in Ws))
    n_in = 2 + len(Ws) + len(bs)
    in_specs = [pl.BlockSpec(memory_space=pltpu.MemorySpace.HBM) if i == 1
                else pl.BlockSpec(memory_space=pltpu.MemorySpace.VMEM)
                for i in range(n_in)]
    out = pl.pallas_call(
        _encoder_kernel,
        out_shape=jax.ShapeDtypeStruct((1, Ws[-1].shape[1]), jnp.float32),
        in_specs=in_specs,
        out_specs=pl.BlockSpec(memory_space=pltpu.MemorySpace.VMEM),
        scratch_shapes=[
            pltpu.VMEM((_N, _N), jnp.float32),
            pltpu.VMEM((_N, _pad128(max_w)), jnp.float32),
            pltpu.VMEM((_N, _pad128(max_w)), jnp.float32),
            pltpu.SemaphoreType.DMA(((_N + _DMA_CHUNK - 1) // _DMA_CHUNK,)),
        ],
        compiler_params=pltpu.CompilerParams(
            vmem_limit_bytes=128 * 1024 * 1024,
        ),
    )(positions, adj, *Ws, *bs2)
    return out.reshape(-1)


# speedup vs baseline: 1.0433x; 1.0250x over previous
import jax
import jax.numpy as jnp
from jax.experimental import pallas as pl
from jax.experimental.pallas import tpu as pltpu

_N = 2562
_CHUNK = 1288
_DMA_CHUNK = 1288


def _pad128(d):
    return ((d + 127) // 128) * 128


def _chunks():
    out = []
    off = 0
    while off < _N:
        out.append((off, min(_CHUNK, _N - off)))
        off += _CHUNK
    return out


def _elu(v):
    return jnp.where(v > 0, v, jnp.exp(jnp.minimum(v, 0.0)) - 1.0)


def _encoder_kernel(*refs):
    pos_ref, adj_hbm = refs[0], refs[1]
    n_layers = (len(refs) - 7) // 2
    w_refs = refs[2:2 + n_layers]
    b_refs = refs[2 + n_layers:2 + 2 * n_layers]
    out_ref = refs[2 + 2 * n_layers]
    adj32 = refs[-4]
    bufs = (refs[-3], refs[-2])
    sems = refs[-1]

    dims = [w.shape for w in w_refs]
    reassoc = [_pad128(din) < _pad128(dout) for din, dout in dims]
    chunks = _chunks()

    dma_chunks = []
    off = 0
    while off < _N:
        dma_chunks.append((off, min(_DMA_CHUNK, _N - off)))
        off += _DMA_CHUNK
    cps = []
    for r, (off, sz) in enumerate(dma_chunks):
        cp = pltpu.make_async_copy(
            adj_hbm.at[pl.ds(off, sz), :],
            adj32.at[pl.ds(off, sz), :],
            sems.at[r])
        cp.start()
        cps.append(cp)

    s0 = jnp.dot(pos_ref[...], w_refs[0][...],
                 preferred_element_type=jnp.float32)
    b0 = b_refs[0][...]

    for r, (off, sz) in enumerate(dma_chunks):
        cps[r].wait()
        a_r = adj32[pl.ds(off, sz), :]
        agg = jnp.dot(a_r, s0, preferred_element_type=jnp.float32)
        xr = _elu(agg + b0)
        if reassoc[1]:
            bufs[1][pl.ds(off, sz), 0:dims[0][1]] = xr
        else:
            s_next = jnp.dot(xr, w_refs[1][...],
                             preferred_element_type=jnp.float32)
            bufs[1][pl.ds(off, sz), 0:dims[1][1]] = s_next

    acc = None
    for i in range(1, n_layers):
        src, dst = bufs[i % 2], bufs[(i + 1) % 2]
        din, dout = dims[i]
        b = b_refs[i][...]
        in_w = din if reassoc[i] else dout
        carry = src[:, 0:in_w]
        w_i = w_refs[i][...]
        if i + 1 < n_layers:
            w_next = w_refs[i + 1][...]
        for off, sz in chunks:
            a_r = adj32[pl.ds(off, sz), :]
            if reassoc[i]:
                h = jnp.dot(a_r, carry, preferred_element_type=jnp.float32)
                agg = jnp.dot(h, w_i, preferred_element_type=jnp.float32)
            else:
                agg = jnp.dot(a_r, carry, preferred_element_type=jnp.float32)
            xr = _elu(agg + b)
            if i + 1 < n_layers:
                if reassoc[i + 1]:
                    dst[pl.ds(off, sz), 0:dout] = xr
                else:
                    s_next = jnp.dot(xr, w_next,
                                     preferred_element_type=jnp.float32)
                    dst[pl.ds(off, sz), 0:dims[i + 1][1]] = s_next
            else:
                m = jnp.max(xr, axis=0, keepdims=True)
                acc = m if acc is None else jnp.maximum(acc, m)
    out_ref[...] = acc


def kernel(positions, adj, Ws, bs):
    bs2 = [b.reshape(1, -1) for b in bs]
    max_w = max(max(d) for d in (w.shape for w in Ws))
    n_in = 2 + len(Ws) + len(bs)
    in_specs = [pl.BlockSpec(memory_space=pltpu.MemorySpace.HBM) if i == 1
                else pl.BlockSpec(memory_space=pltpu.MemorySpace.VMEM)
                for i in range(n_in)]
    out = pl.pallas_call(
        _encoder_kernel,
        out_shape=jax.ShapeDtypeStruct((1, Ws[-1].shape[1]), jnp.float32),
        in_specs=in_specs,
        out_specs=pl.BlockSpec(memory_space=pltpu.MemorySpace.VMEM),
        scratch_shapes=[
            pltpu.VMEM((_N, _N), jnp.float32),
            pltpu.VMEM((_N, _pad128(max_w)), jnp.float32),
            pltpu.VMEM((_N, _pad128(max_w)), jnp.float32),
            pltpu.SemaphoreType.DMA(((_N + _DMA_CHUNK - 1) // _DMA_CHUNK,)),
        ],
        compiler_params=pltpu.CompilerParams(
            vmem_limit_bytes=128 * 1024 * 1024,
        ),
    )(positions, adj, *Ws, *bs2)
    return out.reshape(-1)
